# baseline (speedup 1.0000x reference)
import jax
import jax.numpy as jnp
from jax import lax
from jax.experimental import pallas as pl
from jax.experimental.pallas import tpu as pltpu
from jax.experimental.pallas import tpu_sc as plsc

EMBED_RATIO = 11.313708498984761
D_MODEL = 128
SEQ_LEN = 200
BATCH = 4096

NUM_CORES = 2
NUM_SUBCORES = 16
NUM_WORKERS = NUM_CORES * NUM_SUBCORES
ROWS_PER_WORKER = BATCH * SEQ_LEN // NUM_WORKERS
CHUNK = 128
NQ = ROWS_PER_WORKER // CHUNK
NBUF = 4
LOOKAHEAD = 2
LANES = 16


def _embed_body(inputs_hbm, pe_hbm, table_hbm, out_hbm,
                pe_v, idx_v, rows_v, gsems, wsems):
    wid = lax.axis_index("s") * NUM_CORES + lax.axis_index("c")
    r0 = wid * ROWS_PER_WORKER

    pltpu.sync_copy(pe_hbm, pe_v)
    pltpu.sync_copy(inputs_hbm.at[pl.ds(r0, ROWS_PER_WORKER)], idx_v)

    def issue_gather(q, ph):
        return pltpu.async_copy(
            table_hbm.at[idx_v.at[pl.ds(q * CHUNK, CHUNK)]],
            rows_v.at[ph], gsems[ph])

    for q in range(LOOKAHEAD):
        issue_gather(q, q % NBUF)

    def outer(g):
        for ph in range(NBUF):
            q = g + ph

            phn = (ph + LOOKAHEAD) % NBUF

            @pl.when(q + LOOKAHEAD < NQ)
            def _():
                @pl.when(q >= NBUF - LOOKAHEAD)
                def _():
                    pltpu.make_async_copy(
                        rows_v.at[phn],
                        out_hbm.at[pl.ds(r0, CHUNK)],
                        wsems[phn]).wait()
                issue_gather(q + LOOKAHEAD, phn)

            pltpu.make_async_copy(
                table_hbm.at[idx_v.at[pl.ds(q * CHUNK, CHUNK)]],
                rows_v.at[ph], gsems[ph]).wait()

            buf = rows_v.at[ph]
            s0 = lax.rem(q * CHUNK, SEQ_LEN)
            n1 = lax.min(SEQ_LEN - s0, CHUNK)

            def make_row_body(base):
                def row_body(r, _):
                    for k in range(D_MODEL // LANES):
                        sl = pl.ds(k * LANES, LANES)
                        buf[r, sl] = buf[r, sl] * EMBED_RATIO \
                            + pe_v[base + r, sl]
                    return ()
                return row_body

            lax.fori_loop(0, n1, make_row_body(s0), ())
            lax.fori_loop(n1, CHUNK, make_row_body(s0 - SEQ_LEN), ())

            pltpu.async_copy(
                buf, out_hbm.at[pl.ds(r0 + q * CHUNK, CHUNK)], wsems[ph])

    def outer_body(i, carry):
        outer(i * NBUF)
        return carry

    lax.fori_loop(0, NQ // NBUF, outer_body, ())

    for ph in range(NBUF):
        pltpu.make_async_copy(
            rows_v.at[ph], out_hbm.at[pl.ds(r0, CHUNK)],
            wsems[ph]).wait()


@jax.jit
def kernel(inputs, table, pe):
    inputs_flat = inputs.reshape(BATCH * SEQ_LEN)
    pe2 = pe.reshape(SEQ_LEN, D_MODEL)

    mesh = plsc.VectorSubcoreMesh(
        core_axis_name="c", subcore_axis_name="s",
        num_cores=NUM_CORES, num_subcores=NUM_SUBCORES)

    out = pl.kernel(
        _embed_body,
        out_type=jax.ShapeDtypeStruct((BATCH * SEQ_LEN, D_MODEL),
                                      jnp.float32),
        mesh=mesh,
        scratch_types=[
            pltpu.VMEM((SEQ_LEN, D_MODEL), jnp.float32),
            pltpu.VMEM((ROWS_PER_WORKER,), jnp.int32),
            pltpu.VMEM((NBUF, CHUNK, D_MODEL), jnp.float32),
            [pltpu.SemaphoreType.DMA] * NBUF,
            [pltpu.SemaphoreType.DMA] * NBUF,
        ],
    )(inputs_flat, pe2, table)
    return out.reshape(BATCH, SEQ_LEN, D_MODEL)


# speedup vs baseline: 3.3138x; 3.3138x over previous
import jax
import jax.numpy as jnp
from jax import lax
from jax.experimental import pallas as pl
from jax.experimental.pallas import tpu as pltpu
from jax.experimental.pallas import tpu_sc as plsc

EMBED_RATIO = 11.313708498984761
D_MODEL = 128
SEQ_LEN = 200
BATCH = 4096

NUM_CORES = 2
NUM_SUBCORES = 16
NUM_WORKERS = NUM_CORES * NUM_SUBCORES
SEQ_PER_WORKER = BATCH // NUM_WORKERS
CHUNK = 40
CHUNKS_PER_SEQ = SEQ_LEN // CHUNK
NQ = SEQ_PER_WORKER * CHUNKS_PER_SEQ
NBUF = 2 * CHUNKS_PER_SEQ
LOOKAHEAD = 5
LANES = 16


def _embed_body(inputs_hbm, pe_hbm, table_hbm, out_hbm,
                pe_v, idx_v, rows_v, gsems, wsems):
    wid = lax.axis_index("s") * NUM_CORES + lax.axis_index("c")
    b0 = wid * SEQ_PER_WORKER

    pltpu.sync_copy(pe_hbm, pe_v)
    pltpu.sync_copy(
        inputs_hbm.at[pl.ds(b0 * SEQ_LEN, SEQ_PER_WORKER * SEQ_LEN)], idx_v)

    def issue_gather(q, ph):
        return pltpu.async_copy(
            table_hbm.at[idx_v.at[pl.ds(q * CHUNK, CHUNK)]],
            rows_v.at[ph], gsems[ph])

    for q in range(LOOKAHEAD):
        issue_gather(q, q % NBUF)

    def outer(g):
        for ph in range(NBUF):
            q = g + ph
            h = ph % CHUNKS_PER_SEQ
            t = g // CHUNKS_PER_SEQ + ph // CHUNKS_PER_SEQ

            phn = (ph + LOOKAHEAD) % NBUF
            hn = phn % CHUNKS_PER_SEQ

            @pl.when(q + LOOKAHEAD < NQ)
            def _():
                @pl.when(q >= NBUF - LOOKAHEAD)
                def _():
                    pltpu.make_async_copy(
                        rows_v.at[phn],
                        out_hbm.at[b0, pl.ds(0, CHUNK)],
                        wsems[phn]).wait()
                issue_gather(q + LOOKAHEAD, phn)

            pltpu.make_async_copy(
                table_hbm.at[idx_v.at[pl.ds(q * CHUNK, CHUNK)]],
                rows_v.at[ph], gsems[ph]).wait()

            buf = rows_v.at[ph]

            def row_body(r, _):
                for k in range(D_MODEL // LANES):
                    sl = pl.ds(k * LANES, LANES)
                    buf[r, sl] = buf[r, sl] * EMBED_RATIO \
                        + pe_v[h * CHUNK + r, sl]
                return ()

            lax.fori_loop(0, CHUNK, row_body, ())

            pltpu.async_copy(
                buf, out_hbm.at[b0 + t, pl.ds(h * CHUNK, CHUNK)], wsems[ph])

    def outer_body(i, carry):
        outer(i * NBUF)
        return carry

    lax.fori_loop(0, NQ // NBUF, outer_body, ())

    for ph in range(NBUF):
        pltpu.make_async_copy(
            rows_v.at[ph], out_hbm.at[b0, pl.ds(0, CHUNK)],
            wsems[ph]).wait()


@jax.jit
def kernel(inputs, table, pe):
    inputs_flat = inputs.reshape(BATCH * SEQ_LEN)
    pe2 = pe.reshape(SEQ_LEN, D_MODEL)

    mesh = plsc.VectorSubcoreMesh(
        core_axis_name="c", subcore_axis_name="s",
        num_cores=NUM_CORES, num_subcores=NUM_SUBCORES)

    out = pl.kernel(
        _embed_body,
        out_type=jax.ShapeDtypeStruct((BATCH, SEQ_LEN, D_MODEL),
                                      jnp.float32),
        mesh=mesh,
        scratch_types=[
            pltpu.VMEM((SEQ_LEN, D_MODEL), jnp.float32),
            pltpu.VMEM((SEQ_PER_WORKER * SEQ_LEN,), jnp.int32),
            pltpu.VMEM((NBUF, CHUNK, D_MODEL), jnp.float32),
            [pltpu.SemaphoreType.DMA] * NBUF,
            [pltpu.SemaphoreType.DMA] * NBUF,
        ],
    )(inputs_flat, pe2, table)
    return out
